# baseline (speedup 1.0000x reference)
import functools
import jax
import jax.numpy as jnp
from jax import lax
from jax.experimental import pallas as pl
from jax.experimental.pallas import tpu as pltpu
from jax.experimental.pallas import tpu_sc as plsc

_EPS = 1e-5
_C = 16
_NC = 2
_NS = 16
_NW = _NC * _NS


def kernel(x, pos_table, gamma, beta):
    B, S, D = x.shape
    T = B * S
    xf = x.reshape(T, D)
    tpw = T // _NW
    nlv = D // 16
    mesh = plsc.VectorSubcoreMesh(core_axis_name="c", subcore_axis_name="s")

    @functools.partial(
        pl.kernel,
        mesh=mesh,
        out_type=jax.ShapeDtypeStruct((T, D), jnp.float32),
        compiler_params=pltpu.CompilerParams(needs_layout_passes=False),
        scratch_types=[
            pltpu.VMEM((_C, D), jnp.float32),
            pltpu.VMEM((_C, D), jnp.float32),
            pltpu.VMEM((_C, D), jnp.float32),
            pltpu.VMEM((D,), jnp.float32),
            pltpu.VMEM((D,), jnp.float32),
        ],
    )
    def k(x_hbm, pos_hbm, g_hbm, b_hbm, out_hbm, x_v, p_v, o_v, g_v, b_v):
        wid = lax.axis_index("s") * _NC + lax.axis_index("c")
        base = wid * tpw
        pltpu.sync_copy(g_hbm, g_v)
        pltpu.sync_copy(b_hbm, b_v)

        def chunk_body(i, carry):
            row0 = base + i * _C
            prow0 = lax.rem(row0, S)
            pltpu.sync_copy(x_hbm.at[pl.ds(row0, _C)], x_v)
            pltpu.sync_copy(pos_hbm.at[pl.ds(prow0, _C)], p_v)

            def tok_body(t, tc):
                acc = jnp.zeros((16,), jnp.float32)
                for j in range(nlv):
                    sl = pl.ds(j * 16, 16)
                    v = x_v[t, sl] + p_v[t, sl]
                    x_v[t, sl] = v
                    acc = acc + v
                mean = jnp.sum(acc) * (1.0 / D)
                acc2 = jnp.zeros((16,), jnp.float32)
                for j in range(nlv):
                    sl = pl.ds(j * 16, 16)
                    d = x_v[t, sl] - mean
                    acc2 = acc2 + d * d
                var16 = jnp.full((16,), jnp.sum(acc2) * (1.0 / D) + _EPS,
                                 dtype=jnp.float32)
                iv = lax.bitcast_convert_type(var16, jnp.int32)
                r = lax.bitcast_convert_type(
                    jnp.int32(0x5F3759DF) - lax.shift_right_logical(iv, 1),
                    jnp.float32)
                for _ in range(3):
                    r = r * (1.5 - 0.5 * var16 * r * r)
                for j in range(nlv):
                    sl = pl.ds(j * 16, 16)
                    o_v[t, sl] = (x_v[t, sl] - mean) * r * g_v[sl] + b_v[sl]
                return tc

            lax.fori_loop(0, _C, tok_body, 0)
            pltpu.sync_copy(o_v, out_hbm.at[pl.ds(row0, _C)])
            return carry

        lax.fori_loop(0, tpw // _C, chunk_body, 0)

    out = k(xf, pos_table, gamma, beta)
    return out.reshape(B, S, D)


# speedup vs baseline: 9.1571x; 9.1571x over previous
import jax
import jax.numpy as jnp
from jax.experimental import pallas as pl
from jax.experimental.pallas import tpu as pltpu

_BLK_S = 512
_EPS = 1e-5


def _ln_body(x_ref, pos_ref, g_ref, b_ref, o_ref):
    h = x_ref[...] + pos_ref[...]
    mean = jnp.mean(h, axis=-1, keepdims=True)
    meansq = jnp.mean(h * h, axis=-1, keepdims=True)
    var = meansq - mean * mean
    o_ref[...] = (h - mean) * jax.lax.rsqrt(var + _EPS) * g_ref[...] + b_ref[...]


def kernel(x, pos_table, gamma, beta):
    B, S, D = x.shape
    gamma2 = gamma.reshape(1, 1, D)
    beta2 = beta.reshape(1, 1, D)
    grid = (S // _BLK_S,)
    return pl.pallas_call(
        _ln_body,
        grid=grid,
        in_specs=[
            pl.BlockSpec((B, _BLK_S, D), lambda s: (0, s, 0)),
            pl.BlockSpec((1, _BLK_S, D), lambda s: (0, s, 0)),
            pl.BlockSpec((1, 1, D), lambda s: (0, 0, 0)),
            pl.BlockSpec((1, 1, D), lambda s: (0, 0, 0)),
        ],
        out_specs=pl.BlockSpec((B, _BLK_S, D), lambda s: (0, s, 0)),
        out_shape=jax.ShapeDtypeStruct((B, S, D), x.dtype),
        compiler_params=pltpu.CompilerParams(
            dimension_semantics=("parallel",),
        ),
    )(x, pos_table.reshape(1, S, D), gamma2, beta2)


# speedup vs baseline: 9.1700x; 1.0014x over previous
import jax
import jax.numpy as jnp
from jax.experimental import pallas as pl
from jax.experimental.pallas import tpu as pltpu

_BLK_S = 512
_EPS = 1e-5


def _ln_body(x_ref, pos_ref, g_ref, b_ref, o_ref):
    h = x_ref[...] + pos_ref[...]
    mean = jnp.mean(h, axis=-1, keepdims=True)
    d = h - mean
    var = jnp.mean(d * d, axis=-1, keepdims=True)
    o_ref[...] = d * jax.lax.rsqrt(var + _EPS) * g_ref[...] + b_ref[...]


def kernel(x, pos_table, gamma, beta):
    B, S, D = x.shape
    gamma2 = gamma.reshape(1, 1, D)
    beta2 = beta.reshape(1, 1, D)
    grid = (S // _BLK_S,)
    return pl.pallas_call(
        _ln_body,
        grid=grid,
        in_specs=[
            pl.BlockSpec((B, _BLK_S, D), lambda s: (0, s, 0)),
            pl.BlockSpec((1, _BLK_S, D), lambda s: (0, s, 0)),
            pl.BlockSpec((1, 1, D), lambda s: (0, 0, 0)),
            pl.BlockSpec((1, 1, D), lambda s: (0, 0, 0)),
        ],
        out_specs=pl.BlockSpec((B, _BLK_S, D), lambda s: (0, s, 0)),
        out_shape=jax.ShapeDtypeStruct((B, S, D), x.dtype),
        compiler_params=pltpu.CompilerParams(
            dimension_semantics=("parallel",),
        ),
    )(x, pos_table.reshape(1, S, D), gamma2, beta2)
